# Initial kernel scaffold; baseline (speedup 1.0000x reference)
import jax
import jax.numpy as jnp
from jax.experimental import pallas as pl


def kernel(rois, gt_bboxes_3d, gt_labels_3d, batch_gt_of_rois, batch_gt_label_of_rois, batch_size):
    raise NotImplementedError("write your pallas kernel here")


# speedup vs baseline: 10.4378x; 10.4378x over previous
import functools

import jax
import jax.numpy as jnp
from jax import lax
from jax.experimental import pallas as pl
from jax.experimental.pallas import tpu as pltpu
from jax.experimental.pallas import tpu_sc as plsc

N_CLASSES = 11
REG_FG_THRESH = 0.3
B, R, G = 4, 20000, 128
RPAD = 20480
NW = 32
NROI = B * RPAD // NW
CHUNKS = NROI // 16
W_PER_B = RPAD // NROI


def _sc_body(roi6_hbm, asgn7_hbm, lab_hbm, gt7_hbm, gtlab_hbm,
             out7_hbm, outlab_hbm, mask_hbm,
             roi_v, asgn_v, lab_v, gtflat_v, gtlab_v, gderb_v,
             out_v, outlab_v, mask_v):
    c = lax.axis_index("c")
    s = lax.axis_index("s")
    wid = s * 2 + c
    base = wid * NROI
    b = wid // W_PER_B

    pltpu.sync_copy(roi6_hbm.at[:, pl.ds(base, NROI)], roi_v)
    pltpu.sync_copy(asgn7_hbm.at[:, pl.ds(base, NROI)], asgn_v)
    pltpu.sync_copy(lab_hbm.at[pl.ds(base, NROI)], lab_v)
    pltpu.sync_copy(gt7_hbm.at[b], gtflat_v)
    pltpu.sync_copy(gtlab_hbm.at[b], gtlab_v)

    def bc_body(j, _):
        o = j * 16
        gx = gtflat_v[pl.ds(0 * G + o, 16)]
        gy = gtflat_v[pl.ds(1 * G + o, 16)]
        gz = gtflat_v[pl.ds(2 * G + o, 16)]
        gdx = gtflat_v[pl.ds(3 * G + o, 16)]
        gdy = gtflat_v[pl.ds(4 * G + o, 16)]
        gdz = gtflat_v[pl.ds(5 * G + o, 16)]
        vals = (gx - gdx * 0.5, gy - gdy * 0.5, gz - gdz * 0.5,
                gx + gdx * 0.5, gy + gdy * 0.5, gz + gdz * 0.5,
                (gdx * gdy) * gdz)
        for d, v in enumerate(vals):
            for l in range(16):
                gderb_v[pl.ds((d * G + o + l) * 16, 16)] = jnp.broadcast_to(v[l], (16,))
        return 0

    lax.fori_loop(0, G // 16, bc_body, 0)

    def chunk_body(i, _):
        sl = pl.ds(i * 16, 16)
        rx = roi_v[0, sl]
        ry = roi_v[1, sl]
        rz = roi_v[2, sl]
        rdx = roi_v[3, sl]
        rdy = roi_v[4, sl]
        rdz = roi_v[5, sl]
        rminx = rx - rdx * 0.5
        rminy = ry - rdy * 0.5
        rminz = rz - rdz * 0.5
        rmaxx = rx + rdx * 0.5
        rmaxy = ry + rdy * 0.5
        rmaxz = rz + rdz * 0.5
        volr = (rdx * rdy) * rdz

        def gt_body(g, carry):
            best, bidx = carry
            ix = jnp.maximum(
                jnp.minimum(rmaxx, gderb_v[pl.ds((3 * G + g) * 16, 16)])
                - jnp.maximum(rminx, gderb_v[pl.ds((0 * G + g) * 16, 16)]), 0.0)
            iy = jnp.maximum(
                jnp.minimum(rmaxy, gderb_v[pl.ds((4 * G + g) * 16, 16)])
                - jnp.maximum(rminy, gderb_v[pl.ds((1 * G + g) * 16, 16)]), 0.0)
            iz = jnp.maximum(
                jnp.minimum(rmaxz, gderb_v[pl.ds((5 * G + g) * 16, 16)])
                - jnp.maximum(rminz, gderb_v[pl.ds((2 * G + g) * 16, 16)]), 0.0)
            inter = (ix * iy) * iz
            iou = inter / (((volr + gderb_v[pl.ds((6 * G + g) * 16, 16)]) - inter) + 1e-8)
            better = iou > best
            best = jnp.where(better, iou, best)
            bidx = jnp.where(better, g, bidx)
            return best, bidx

        best, bidx = lax.fori_loop(
            0, G, gt_body,
            (jnp.full((16,), -1.0, jnp.float32), jnp.zeros((16,), jnp.int32)))

        lab = lab_v[sl]
        pos = lab >= 0
        stage2 = jnp.logical_and(jnp.logical_not(pos), best >= REG_FG_THRESH)
        zero = jnp.zeros((16,), jnp.float32)
        for comp in range(6):
            a = plsc.load_gather(gtflat_v, [bidx + comp * G])
            out_v[comp, sl] = jnp.where(pos, asgn_v[comp, sl],
                                        jnp.where(stage2, a, zero))
        ayaw = -plsc.load_gather(gtflat_v, [bidx + 6 * G])
        out_v[6, sl] = jnp.where(pos, -asgn_v[6, sl],
                                 jnp.where(stage2, ayaw, zero))
        alab = plsc.load_gather(gtlab_v, [bidx])
        outlab_v[sl] = jnp.where(pos, lab,
                                 jnp.where(stage2, alab,
                                           jnp.full((16,), N_CLASSES, jnp.int32)))
        mask_v[sl] = jnp.logical_or(pos, stage2).astype(jnp.float32)
        return 0

    lax.fori_loop(0, CHUNKS, chunk_body, 0)

    pltpu.sync_copy(out_v, out7_hbm.at[:, pl.ds(base, NROI)])
    pltpu.sync_copy(outlab_v, outlab_hbm.at[pl.ds(base, NROI)])
    pltpu.sync_copy(mask_v, mask_hbm.at[pl.ds(base, NROI)])


@jax.jit
def _run(roi6, asgn7, lab, gt7, gtlab):
    mesh = plsc.VectorSubcoreMesh(core_axis_name="c", subcore_axis_name="s",
                                  num_cores=2, num_subcores=16)
    f = pl.kernel(
        _sc_body,
        out_type=[
            jax.ShapeDtypeStruct((7, B * RPAD), jnp.float32),
            jax.ShapeDtypeStruct((B * RPAD,), jnp.int32),
            jax.ShapeDtypeStruct((B * RPAD,), jnp.float32),
        ],
        mesh=mesh,
        compiler_params=pltpu.CompilerParams(needs_layout_passes=False),
        scratch_types=[
            pltpu.VMEM((6, NROI), jnp.float32),
            pltpu.VMEM((7, NROI), jnp.float32),
            pltpu.VMEM((NROI,), jnp.int32),
            pltpu.VMEM((7 * G,), jnp.float32),
            pltpu.VMEM((G,), jnp.int32),
            pltpu.VMEM((7 * G * 16,), jnp.float32),
            pltpu.VMEM((7, NROI), jnp.float32),
            pltpu.VMEM((NROI,), jnp.int32),
            pltpu.VMEM((NROI,), jnp.float32),
        ],
    )
    return f(roi6, asgn7, lab, gt7, gtlab)


def kernel(rois, gt_bboxes_3d, gt_labels_3d, batch_gt_of_rois,
           batch_gt_label_of_rois, batch_size):
    pad = RPAD - R
    roi6 = jnp.pad(rois[..., :6], ((0, 0), (0, pad), (0, 0)))
    roi6 = roi6.transpose(2, 0, 1).reshape(6, B * RPAD)
    asgn7 = jnp.pad(batch_gt_of_rois, ((0, 0), (0, pad), (0, 0)))
    asgn7 = asgn7.transpose(2, 0, 1).reshape(7, B * RPAD)
    lab = jnp.pad(batch_gt_label_of_rois, ((0, 0), (0, pad)),
                  constant_values=-1).reshape(B * RPAD)
    gt7 = gt_bboxes_3d.transpose(0, 2, 1).reshape(B, 7 * G)
    gtlab = gt_labels_3d

    out7, outlab, maskf = _run(roi6, asgn7, lab, gt7, gtlab)

    gt_out = out7.reshape(7, B, RPAD)[:, :, :R].transpose(1, 2, 0)
    lab_out = outlab.reshape(B, RPAD)[:, :R]
    mask_out = maskf.reshape(B, RPAD)[:, :R]
    return rois, gt_out, lab_out, mask_out
